# baseline (speedup 1.0000x reference)
import functools

import jax
import jax.numpy as jnp
from jax.experimental import pallas as pl
from jax.experimental.pallas import tpu as pltpu

B, T, C, H = 1, 2048, 1024, 16
DH = C // H
CHUNK, TOP_B, LOCAL, RDIM = 64, 8, 256, 32
NC = T // CHUNK
QBLK = 256
KBLK = 256
NQB = T // QBLK
SCALE = 1.0 / (DH ** 0.5)
NEG = -1e30


def _topk_kernel(sc_ref, sel_ref):
    scores = sc_ref[...]
    lo_r = jax.lax.broadcasted_iota(jnp.int32, (NC, NC), 0)
    lo_c = jax.lax.broadcasted_iota(jnp.int32, (NC, NC), 1)
    Ltri = jnp.where(lo_r < lo_c, jnp.float32(1.0), 0.0)
    sel = jnp.zeros((T, NC), jnp.float32)
    s = scores
    for _ in range(TOP_B):
        m = jnp.max(s, axis=-1, keepdims=True)
        ismax = (s == m).astype(jnp.float32)
        prefix = jax.lax.dot_general(ismax, Ltri, (((1,), (0,)), ((), ())),
                                     preferred_element_type=jnp.float32)
        first = (ismax > 0.5) & (prefix < 0.5)
        sel = jnp.where(first, 1.0, sel)
        s = jnp.where(first, NEG, s)
    sel_ref[...] = sel


def _qkv_kernel(x_ref, w_ref, b_ref, o_ref):
    acc = jax.lax.dot_general(
        x_ref[...].astype(jnp.bfloat16), w_ref[...],
        (((1,), (0,)), ((), ())),
        preferred_element_type=jnp.float32) + b_ref[...]
    o_ref[...] = acc.astype(jnp.bfloat16)


def _flash_kernel(q_ref, k_ref, v_ref, sel_ref, o_ref, bias_ref, acc_ref):
    h = pl.program_id(0)
    i = pl.program_id(1)
    kj = pl.program_id(2)

    @pl.when((h == 0) & (kj == 0))
    def _build_bias():
        sel_blk = sel_ref[...]
        rowc = jax.lax.broadcasted_iota(jnp.int32, (NC, T), 0)
        colj = jax.lax.broadcasted_iota(jnp.int32, (NC, T), 1)
        E = jnp.where(colj // CHUNK == rowc, jnp.float32(1.0), 0.0)
        retrieved = jax.lax.dot_general(
            sel_blk, E, (((1,), (0,)), ((), ())),
            preferred_element_type=jnp.float32) > 0.5
        row_i = (i * QBLK
                 + jax.lax.broadcasted_iota(jnp.int32, (QBLK, T), 0))
        col_j = jax.lax.broadcasted_iota(jnp.int32, (QBLK, T), 1)
        allowed = (col_j <= row_i) & (((row_i - col_j) < LOCAL) | retrieved)
        bias_ref[pl.ds(i * QBLK, QBLK), :] = jnp.where(allowed, 0.0, NEG)

    @pl.when(kj <= i)
    def _attend():
        q = q_ref[0] * jnp.bfloat16(SCALE)
        s = jax.lax.dot_general(q, k_ref[0], (((1,), (1,)), ((), ())),
                                preferred_element_type=jnp.float32)
        s = s + bias_ref[pl.ds(i * QBLK, QBLK), pl.ds(kj * KBLK, KBLK)]
        p = jnp.exp(s)
        pv = jax.lax.dot_general(
            p.astype(jnp.bfloat16), v_ref[0], (((1,), (0,)), ((), ())),
            preferred_element_type=jnp.float32)
        acc = jnp.where(kj == 0, pv, acc_ref[...] + pv)
        acc_ref[...] = acc

        @pl.when(kj == i)
        def _finish():
            o_ref[0] = acc[:, :DH] / acc[:, DH:DH + 1]


def _oproj_kernel(y_ref, w_ref, b_ref, o_ref):
    o_ref[...] = (jax.lax.dot_general(
        y_ref[...].astype(jnp.bfloat16), w_ref[...],
        (((1,), (0,)), ((), ())),
        preferred_element_type=jnp.float32) + b_ref[...])


@functools.partial(jax.jit, static_argnames=())
def kernel(x, W_router, b_router, W_gate, b_gate, W_qkv, b_qkv, W_o, b_o):
    x2 = x.reshape(T, C)

    chunk_means = x.reshape(B, NC, CHUNK, C).mean(axis=2)
    routing_embeds = chunk_means @ W_router + b_router
    q_rout = x @ W_gate + b_gate
    q_rout = q_rout / jnp.maximum(
        jnp.linalg.norm(q_rout, axis=-1, keepdims=True), 1e-12)
    r_emb = routing_embeds / jnp.maximum(
        jnp.linalg.norm(routing_embeds, axis=-1, keepdims=True), 1e-12)
    routing_scores = jnp.einsum('btd,bnd->btn', q_rout, r_emb)

    sel = pl.pallas_call(
        _topk_kernel,
        out_shape=jax.ShapeDtypeStruct((T, NC), jnp.float32),
    )(routing_scores.reshape(T, NC))

    qkv = pl.pallas_call(
        _qkv_kernel,
        grid=(6,),
        in_specs=[
            pl.BlockSpec((T, C), lambda i: (0, 0)),
            pl.BlockSpec((C, 512), lambda i: (0, i)),
            pl.BlockSpec((1, 512), lambda i: (0, i)),
        ],
        out_specs=pl.BlockSpec((T, 512), lambda i: (0, i)),
        out_shape=jax.ShapeDtypeStruct((T, 3 * C), jnp.bfloat16),
    )(x2, W_qkv.astype(jnp.bfloat16), b_qkv.reshape(1, 3 * C))

    q = qkv[:, :C].reshape(T, H, DH).transpose(1, 0, 2)
    k = qkv[:, C:2 * C].reshape(T, H, DH).transpose(1, 0, 2)
    v = qkv[:, 2 * C:].reshape(T, H, DH).transpose(1, 0, 2)
    v = jnp.concatenate(
        [v, jnp.ones((H, T, 1), jnp.bfloat16),
         jnp.zeros((H, T, DH - 1), jnp.bfloat16)], axis=-1)

    y3 = pl.pallas_call(
        _flash_kernel,
        grid=(H, NQB, NQB),
        in_specs=[
            pl.BlockSpec((1, QBLK, DH), lambda h, i, kj: (h, i, 0)),
            pl.BlockSpec((1, KBLK, DH), lambda h, i, kj: (h, kj, 0)),
            pl.BlockSpec((1, KBLK, 2 * DH), lambda h, i, kj: (h, kj, 0)),
            pl.BlockSpec((QBLK, NC), lambda h, i, kj: (i, 0)),
        ],
        out_specs=pl.BlockSpec((1, QBLK, DH), lambda h, i, kj: (h, i, 0)),
        out_shape=jax.ShapeDtypeStruct((H, T, DH), jnp.float32),
        scratch_shapes=[pltpu.VMEM((T, T), jnp.float32),
                        pltpu.VMEM((QBLK, 2 * DH), jnp.float32)],
    )(q, k, v, sel)

    y = y3.transpose(1, 0, 2).reshape(T, C)

    out = pl.pallas_call(
        _oproj_kernel,
        grid=(NQB,),
        in_specs=[
            pl.BlockSpec((QBLK, C), lambda i: (i, 0)),
            pl.BlockSpec((C, C), lambda i: (0, 0)),
            pl.BlockSpec((1, C), lambda i: (0, 0)),
        ],
        out_specs=pl.BlockSpec((QBLK, C), lambda i: (i, 0)),
        out_shape=jax.ShapeDtypeStruct((T, C), jnp.float32),
    )(y, W_o.astype(jnp.bfloat16), b_o.reshape(1, C))

    return out.reshape(B, T, C)


# speedup vs baseline: 3.5206x; 3.5206x over previous
import functools

import jax
import jax.numpy as jnp
from jax.experimental import pallas as pl
from jax.experimental.pallas import tpu as pltpu

B, T, C, H = 1, 2048, 1024, 16
DH = C // H
CHUNK, TOP_B, LOCAL, RDIM = 64, 8, 256, 32
NC = T // CHUNK
QBLK = 256
KBLK = 256
NQB = T // QBLK
SCALE = 1.0 / (DH ** 0.5)
NEG = -1e30


def _topk_kernel(sc_ref, sel_ref):
    scores = sc_ref[...]
    lo_r = jax.lax.broadcasted_iota(jnp.int32, (NC, NC), 0)
    lo_c = jax.lax.broadcasted_iota(jnp.int32, (NC, NC), 1)
    Ltri = jnp.where(lo_r < lo_c, jnp.float32(1.0), 0.0)
    sel = jnp.zeros((T, NC), jnp.float32)
    s = scores
    for _ in range(TOP_B):
        m = jnp.max(s, axis=-1, keepdims=True)
        ismax = (s == m).astype(jnp.float32)
        prefix = jax.lax.dot_general(ismax, Ltri, (((1,), (0,)), ((), ())),
                                     preferred_element_type=jnp.float32)
        first = (ismax > 0.5) & (prefix < 0.5)
        sel = jnp.where(first, 1.0, sel)
        s = jnp.where(first, NEG, s)
    sel_ref[...] = sel


def _qkv_kernel(x_ref, w_ref, b_ref, o_ref):
    acc = jax.lax.dot_general(
        x_ref[...].astype(jnp.bfloat16), w_ref[...],
        (((1,), (0,)), ((), ())),
        preferred_element_type=jnp.float32) + b_ref[...]
    o_ref[...] = acc.astype(jnp.bfloat16)


def _flash_kernel(q_ref, k_ref, v_ref, sel_ref, o_ref, bias_ref):
    h = pl.program_id(0)

    @pl.when(h == 0)
    def _build_bias():
        sel_all = sel_ref[...]
        rowc = jax.lax.broadcasted_iota(jnp.int32, (NC, T), 0)
        colj = jax.lax.broadcasted_iota(jnp.int32, (NC, T), 1)
        E = jnp.where(colj // CHUNK == rowc, jnp.float32(1.0), 0.0)
        retrieved = jax.lax.dot_general(
            sel_all, E, (((1,), (0,)), ((), ())),
            preferred_element_type=jnp.float32) > 0.5
        row_i = jax.lax.broadcasted_iota(jnp.int32, (T, T), 0)
        col_j = jax.lax.broadcasted_iota(jnp.int32, (T, T), 1)
        allowed = (col_j <= row_i) & (((row_i - col_j) < LOCAL) | retrieved)
        bias_ref[...] = jnp.where(allowed, 0.0, NEG)

    q_all = q_ref[0] * jnp.bfloat16(SCALE)
    for i in range(NQB):
        q_i = q_all[i * QBLK:(i + 1) * QBLK]
        acc = None
        for kj in range(i + 1):
            k_blk = k_ref[0, kj * KBLK:(kj + 1) * KBLK, :]
            v_blk = v_ref[0, kj * KBLK:(kj + 1) * KBLK, :]
            s = jax.lax.dot_general(q_i, k_blk, (((1,), (1,)), ((), ())),
                                    preferred_element_type=jnp.float32)
            s = s + bias_ref[i * QBLK:(i + 1) * QBLK,
                             kj * KBLK:(kj + 1) * KBLK]
            p = jnp.exp(s)
            pv = jax.lax.dot_general(
                p.astype(jnp.bfloat16), v_blk, (((1,), (0,)), ((), ())),
                preferred_element_type=jnp.float32)
            acc = pv if kj == 0 else acc + pv
        o_ref[0, i * QBLK:(i + 1) * QBLK, :] = (
            acc[:, :DH] / acc[:, DH:DH + 1])


def _oproj_kernel(y_ref, w_ref, b_ref, o_ref):
    o_ref[...] = (jax.lax.dot_general(
        y_ref[...].astype(jnp.bfloat16), w_ref[...],
        (((1,), (0,)), ((), ())),
        preferred_element_type=jnp.float32) + b_ref[...])


@functools.partial(jax.jit, static_argnames=())
def kernel(x, W_router, b_router, W_gate, b_gate, W_qkv, b_qkv, W_o, b_o):
    x2 = x.reshape(T, C)

    chunk_means = x.reshape(B, NC, CHUNK, C).mean(axis=2)
    routing_embeds = chunk_means @ W_router + b_router
    q_rout = x @ W_gate + b_gate
    q_rout = q_rout / jnp.maximum(
        jnp.linalg.norm(q_rout, axis=-1, keepdims=True), 1e-12)
    r_emb = routing_embeds / jnp.maximum(
        jnp.linalg.norm(routing_embeds, axis=-1, keepdims=True), 1e-12)
    routing_scores = jnp.einsum('btd,bnd->btn', q_rout, r_emb)

    sel = pl.pallas_call(
        _topk_kernel,
        out_shape=jax.ShapeDtypeStruct((T, NC), jnp.float32),
    )(routing_scores.reshape(T, NC))

    qkv = pl.pallas_call(
        _qkv_kernel,
        grid=(6,),
        in_specs=[
            pl.BlockSpec((T, C), lambda i: (0, 0)),
            pl.BlockSpec((C, 512), lambda i: (0, i)),
            pl.BlockSpec((1, 512), lambda i: (0, i)),
        ],
        out_specs=pl.BlockSpec((T, 512), lambda i: (0, i)),
        out_shape=jax.ShapeDtypeStruct((T, 3 * C), jnp.bfloat16),
    )(x2, W_qkv.astype(jnp.bfloat16), b_qkv.reshape(1, 3 * C))

    q = qkv[:, :C].reshape(T, H, DH).transpose(1, 0, 2)
    k = qkv[:, C:2 * C].reshape(T, H, DH).transpose(1, 0, 2)
    v = qkv[:, 2 * C:].reshape(T, H, DH).transpose(1, 0, 2)
    v = jnp.concatenate(
        [v, jnp.ones((H, T, 1), jnp.bfloat16),
         jnp.zeros((H, T, DH - 1), jnp.bfloat16)], axis=-1)

    y3 = pl.pallas_call(
        _flash_kernel,
        grid=(H,),
        in_specs=[
            pl.BlockSpec((1, T, DH), lambda h: (h, 0, 0)),
            pl.BlockSpec((1, T, DH), lambda h: (h, 0, 0)),
            pl.BlockSpec((1, T, 2 * DH), lambda h: (h, 0, 0)),
            pl.BlockSpec((T, NC), lambda h: (0, 0)),
        ],
        out_specs=pl.BlockSpec((1, T, DH), lambda h: (h, 0, 0)),
        out_shape=jax.ShapeDtypeStruct((H, T, DH), jnp.float32),
        scratch_shapes=[pltpu.VMEM((T, T), jnp.float32)],
    )(q, k, v, sel)

    y = y3.transpose(1, 0, 2).reshape(T, C)

    out = pl.pallas_call(
        _oproj_kernel,
        grid=(NQB,),
        in_specs=[
            pl.BlockSpec((QBLK, C), lambda i: (i, 0)),
            pl.BlockSpec((C, C), lambda i: (0, 0)),
            pl.BlockSpec((1, C), lambda i: (0, 0)),
        ],
        out_specs=pl.BlockSpec((QBLK, C), lambda i: (i, 0)),
        out_shape=jax.ShapeDtypeStruct((T, C), jnp.float32),
    )(y, W_o.astype(jnp.bfloat16), b_o.reshape(1, C))

    return out.reshape(B, T, C)
